# baseline (speedup 1.0000x reference)
import functools

import jax
import jax.numpy as jnp
from jax import lax
from jax.experimental import pallas as pl
from jax.experimental.pallas import tpu as pltpu
from jax.experimental.pallas import tpu_sc as plsc

NUM_CORES = 2
NUM_SUBCORES = 16
NUM_WORKERS = NUM_CORES * NUM_SUBCORES
LANES = 16
CHUNK = 128


def _gmf_body(factors, bpw, user_ids_hbm, item_ids_hbm, ut_hbm, it_hbm,
              out_hbm, uids_v, iids_v, ucidx0_v, ucidx1_v, icidx0_v,
              icidx1_v, ucol_v, icol_v, uchunks_v, ichunks_v, out_v, sem):
    ucidx = (ucidx0_v, ucidx1_v)
    icidx = (icidx0_v, icidx1_v)
    rows_per_chunk = CHUNK // factors
    half = bpw // 2
    wid = lax.axis_index("s") * NUM_CORES + lax.axis_index("c")
    base = wid * bpw

    pltpu.sync_copy(user_ids_hbm.at[pl.ds(base, bpw)], uids_v)
    pltpu.sync_copy(item_ids_hbm.at[pl.ds(base, bpw)], iids_v)

    for h in range(2):
        def prep(j, carry, h=h):
            k = h * (half // LANES) + j
            uid = uids_v[pl.ds(k * LANES, LANES)]
            iid = iids_v[pl.ds(k * LANES, LANES)]
            ucidx[h][pl.ds(j * LANES, LANES)] = uid // rows_per_chunk
            icidx[h][pl.ds(j * LANES, LANES)] = iid // rows_per_chunk
            ucol_v[pl.ds(k * LANES, LANES)] = (uid % rows_per_chunk) * factors
            icol_v[pl.ds(k * LANES, LANES)] = (iid % rows_per_chunk) * factors
            return carry

        lax.fori_loop(0, half // LANES, prep, 0)

    lane = lax.iota(jnp.int32, LANES)

    for h in range(2):
        cu = pltpu.async_copy(ut_hbm.at[ucidx[h]], uchunks_v, sem)
        ci = pltpu.async_copy(it_hbm.at[icidx[h]], ichunks_v, sem)
        cu.wait()
        ci.wait()

        def group(g, carry, h=h):
            rows = g * LANES + lane
            ucol = ucol_v[pl.ds(h * half + g * LANES, LANES)]
            icol = icol_v[pl.ds(h * half + g * LANES, LANES)]
            acc = jnp.zeros((LANES,), jnp.float32)
            for f in range(factors):
                uv = plsc.load_gather(uchunks_v, [rows, ucol + f])
                iv = plsc.load_gather(ichunks_v, [rows, icol + f])
                acc = acc + uv * iv
            out_v[pl.ds(h * half + g * LANES, LANES)] = (
                1.0 / (1.0 + jnp.exp(-acc)))
            return carry

        lax.fori_loop(0, half // LANES, group, 0)

    pltpu.sync_copy(out_v, out_hbm.at[pl.ds(base, bpw)])


def kernel(user_ids, item_ids, user_table, item_table):
    batch = user_ids.shape[0]
    factors = user_table.shape[1]
    bpw = batch // NUM_WORKERS
    half = bpw // 2

    ut = user_table.reshape(-1, CHUNK)
    it = item_table.reshape(-1, CHUNK)

    mesh = plsc.VectorSubcoreMesh(
        core_axis_name="c", subcore_axis_name="s",
        num_cores=NUM_CORES, num_subcores=NUM_SUBCORES)

    run = pl.kernel(
        functools.partial(_gmf_body, factors, bpw),
        out_type=jax.ShapeDtypeStruct((batch,), jnp.float32),
        mesh=mesh,
        scratch_types=[
            pltpu.VMEM((bpw,), jnp.int32),
            pltpu.VMEM((bpw,), jnp.int32),
            pltpu.VMEM((half,), jnp.int32),
            pltpu.VMEM((half,), jnp.int32),
            pltpu.VMEM((half,), jnp.int32),
            pltpu.VMEM((half,), jnp.int32),
            pltpu.VMEM((bpw,), jnp.int32),
            pltpu.VMEM((bpw,), jnp.int32),
            pltpu.VMEM((half, CHUNK), jnp.float32),
            pltpu.VMEM((half, CHUNK), jnp.float32),
            pltpu.VMEM((bpw,), jnp.float32),
            pltpu.SemaphoreType.DMA,
        ],
        compiler_params=pltpu.CompilerParams(needs_layout_passes=False),
    )
    return run(user_ids.astype(jnp.int32), item_ids.astype(jnp.int32),
               ut, it)


# speedup vs baseline: 2.4755x; 2.4755x over previous
import functools

import jax
import jax.numpy as jnp
from jax import lax
from jax.experimental import pallas as pl
from jax.experimental.pallas import tpu as pltpu
from jax.experimental.pallas import tpu_sc as plsc

NUM_CORES = 2
NUM_SUBCORES = 16
NUM_WORKERS = NUM_CORES * NUM_SUBCORES
LANES = 16


def _gmf_body(factors, bpw, user_ids_hbm, item_ids_hbm, ut_hbm, it_hbm,
              out_hbm, uids_v, iids_v, urows_v, irows_v, out_v, sem):
    wid = lax.axis_index("s") * NUM_CORES + lax.axis_index("c")
    base = wid * bpw

    pltpu.sync_copy(user_ids_hbm.at[pl.ds(base, bpw)], uids_v)
    pltpu.sync_copy(item_ids_hbm.at[pl.ds(base, bpw)], iids_v)

    lane = lax.iota(jnp.int32, LANES)
    half = bpw // 2

    for h in range(2):
        def fetch(g, carry, h=h):
            uidv = uids_v[pl.ds(h * half + g * LANES, LANES)]
            iidv = iids_v[pl.ds(h * half + g * LANES, LANES)]
            for l in range(LANES):
                uid = jnp.sum(jnp.where(lane == l, uidv, 0))
                iid = jnp.sum(jnp.where(lane == l, iidv, 0))
                row = g * LANES + l
                pltpu.async_copy(
                    ut_hbm.at[pl.ds(uid // 8, 1), pl.ds(uid % 8, 1), :],
                    urows_v.at[pl.ds(row // 8, 1), pl.ds(row % 8, 1), :],
                    sem)
                pltpu.async_copy(
                    it_hbm.at[pl.ds(iid // 8, 1), pl.ds(iid % 8, 1), :],
                    irows_v.at[pl.ds(row // 8, 1), pl.ds(row % 8, 1), :],
                    sem)
            return carry

        lax.fori_loop(0, half // LANES, fetch, 0)

        pltpu.make_async_copy(ut_hbm.at[pl.ds(0, half // 8)], urows_v,
                              sem).wait()
        pltpu.make_async_copy(it_hbm.at[pl.ds(0, half // 8)], irows_v,
                              sem).wait()

        def group(g, carry, h=h):
            rows = g * LANES + lane
            acc = jnp.zeros((LANES,), jnp.float32)
            for f in range(factors):
                col = jnp.full((LANES,), f, jnp.int32)
                uv = plsc.load_gather(urows_v, [rows // 8, rows % 8, col])
                iv = plsc.load_gather(irows_v, [rows // 8, rows % 8, col])
                acc = acc + uv * iv
            out_v[pl.ds(h * half + g * LANES, LANES)] = (
                1.0 / (1.0 + jnp.exp(-acc)))
            return carry

        lax.fori_loop(0, half // LANES, group, 0)

    pltpu.sync_copy(out_v, out_hbm.at[pl.ds(base, bpw)])


def kernel(user_ids, item_ids, user_table, item_table):
    batch = user_ids.shape[0]
    factors = user_table.shape[1]
    bpw = batch // NUM_WORKERS

    mesh = plsc.VectorSubcoreMesh(
        core_axis_name="c", subcore_axis_name="s",
        num_cores=NUM_CORES, num_subcores=NUM_SUBCORES)

    run = pl.kernel(
        functools.partial(_gmf_body, factors, bpw),
        out_type=jax.ShapeDtypeStruct((batch,), jnp.float32),
        mesh=mesh,
        scratch_types=[
            pltpu.VMEM((bpw,), jnp.int32),
            pltpu.VMEM((bpw,), jnp.int32),
            pltpu.VMEM((bpw // 16, 8, factors), jnp.float32),
            pltpu.VMEM((bpw // 16, 8, factors), jnp.float32),
            pltpu.VMEM((bpw,), jnp.float32),
            pltpu.SemaphoreType.DMA,
        ],
        compiler_params=pltpu.CompilerParams(needs_layout_passes=False),
    )
    return run(user_ids.astype(jnp.int32), item_ids.astype(jnp.int32),
               user_table.reshape(-1, 8, factors),
               item_table.reshape(-1, 8, factors))
